# baseline (speedup 1.0000x reference)
import functools

import jax
import jax.numpy as jnp
from jax import lax
from jax.experimental import pallas as pl
from jax.experimental.pallas import tpu as pltpu
from jax.experimental.pallas import tpu_sc as plsc

B, S, D, E, FF = 2, 2048, 1024, 8, 2048
K = 512
SB = 512
FFB = 512
NC, NS, L = 2, 16, 16
KPAD = K + 64
GCHUNK = 64


def _gating_body(x_ref, gw_ref, out_ref):
    logits_t = lax.dot_general(
        gw_ref[...], x_ref[0], (((0,), (1,)), ((), ())),
        preferred_element_type=jnp.float32)
    m = jnp.max(logits_t, axis=0, keepdims=True)
    ex = jnp.exp(logits_t - m)
    out_ref[0] = ex / jnp.sum(ex, axis=0, keepdims=True)


def _gating(x, gate_weight):
    return pl.pallas_call(
        _gating_body,
        grid=(B, S // SB),
        in_specs=[
            pl.BlockSpec((1, SB, D), lambda b, s: (b, s, 0)),
            pl.BlockSpec((D, E), lambda b, s: (0, 0)),
        ],
        out_specs=pl.BlockSpec((1, E, SB), lambda b, s: (b, 0, s)),
        out_shape=jax.ShapeDtypeStruct((B, E, S), jnp.float32),
    )(x, gate_weight)


def _route_gather_body(aff_f_hbm, aff_i_hbm, x2d_hbm, gidx_hbm, gates_hbm,
                       xsel_hbm, row_vf, row_vi, idx_v, gv_v, buf_v, sem):
    c = lax.axis_index("c")
    s = lax.axis_index("s")
    wid = s * NC + c

    @pl.when(wid < B * E)
    def _work():
        b = wid // E
        e = wid % E
        pltpu.sync_copy(aff_f_hbm.at[b, e], row_vf)
        pltpu.sync_copy(aff_i_hbm.at[b, e], row_vi)

        nvec = S // L

        def _count_ge(t_vec):
            def cbody(j, acc):
                kx = row_vi[pl.ds(j * L, L)]
                return acc + jnp.where(kx >= t_vec, 1, 0)
            acc = lax.fori_loop(0, nvec, cbody, jnp.zeros((L,), jnp.int32))
            return jnp.sum(acc)

        def bbody(i, t):
            cand = t | lax.shift_left(jnp.int32(1), jnp.int32(30) - i)
            cnt = _count_ge(jnp.full((L,), cand, jnp.int32))
            return jnp.where(cnt >= K, cand, t)

        t = lax.fori_loop(0, 31, bbody, jnp.int32(0))
        t_vec = jnp.full((L,), t, jnp.int32)

        def pbody(j, off):
            v = row_vf[pl.ds(j * L, L)]
            kx = row_vi[pl.ds(j * L, L)]
            m = kx >= t_vec
            plsc.store_compressed(gv_v.at[pl.ds(off, L)], v, mask=m)
            gi = j * L + lax.iota(jnp.int32, L) + b * S
            plsc.store_compressed(idx_v.at[pl.ds(off, L)], gi, mask=m)
            return off + jnp.sum(jnp.where(m, 1, 0))

        lax.fori_loop(0, nvec, pbody, jnp.int32(0))

        pltpu.sync_copy(idx_v.at[pl.ds(0, K)], gidx_hbm.at[e, pl.ds(b * K, K)])
        pltpu.sync_copy(gv_v.at[pl.ds(0, K)], gates_hbm.at[e, pl.ds(b * K, K)])

        def gbody(ci, carry):
            idxs = idx_v.at[pl.ds(ci * GCHUNK, GCHUNK)]
            pltpu.async_copy(x2d_hbm.at[idxs], buf_v, sem).wait()
            pltpu.sync_copy(
                buf_v, xsel_hbm.at[e, pl.ds(b * K + ci * GCHUNK, GCHUNK)])
            return carry

        lax.fori_loop(0, K // GCHUNK, gbody, jnp.int32(0))


def _route_gather(aff_t, x2d):
    mesh = plsc.VectorSubcoreMesh(core_axis_name="c", subcore_axis_name="s")
    f = pl.kernel(
        _route_gather_body,
        out_type=(
            jax.ShapeDtypeStruct((E, B * K), jnp.int32),
            jax.ShapeDtypeStruct((E, B * K), jnp.float32),
            jax.ShapeDtypeStruct((E, B * K, D), jnp.float32),
        ),
        mesh=mesh,
        compiler_params=pltpu.CompilerParams(needs_layout_passes=False),
        scratch_types=[
            pltpu.VMEM((S,), jnp.float32),
            pltpu.VMEM((S,), jnp.int32),
            pltpu.VMEM((KPAD,), jnp.int32),
            pltpu.VMEM((KPAD,), jnp.float32),
            pltpu.VMEM((GCHUNK, D), jnp.float32),
            pltpu.SemaphoreType.DMA,
        ],
    )
    aff_i = lax.bitcast_convert_type(aff_t, jnp.int32)
    return f(aff_t, aff_i, x2d)


def _swiglu_body(xsel_ref, gates_ref, wg_ref, wu_ref, wd_ref, out_ref):
    a = xsel_ref[0]
    g = lax.dot_general(a, wg_ref[0], (((1,), (1,)), ((), ())),
                        preferred_element_type=jnp.float32)
    u = lax.dot_general(a, wu_ref[0], (((1,), (1,)), ((), ())),
                        preferred_element_type=jnp.float32)
    h = g * jax.nn.sigmoid(g) * u
    h = h * gates_ref[0, 0][:, None]
    part = lax.dot_general(h, wd_ref[0], (((1,), (1,)), ((), ())),
                           preferred_element_type=jnp.float32)

    @pl.when(pl.program_id(1) == 0)
    def _():
        out_ref[0] = part

    @pl.when(pl.program_id(1) != 0)
    def _():
        out_ref[0] += part


def _swiglu(x_sel, gates, Wg, Wu, Wd):
    return pl.pallas_call(
        _swiglu_body,
        grid=(E, FF // FFB),
        in_specs=[
            pl.BlockSpec((1, B * K, D), lambda e, f: (e, 0, 0)),
            pl.BlockSpec((1, 1, B * K), lambda e, f: (e, 0, 0)),
            pl.BlockSpec((1, FFB, D), lambda e, f: (e, f, 0)),
            pl.BlockSpec((1, FFB, D), lambda e, f: (e, f, 0)),
            pl.BlockSpec((1, D, FFB), lambda e, f: (e, 0, f)),
        ],
        out_specs=pl.BlockSpec((1, B * K, D), lambda e, f: (e, 0, 0)),
        out_shape=jax.ShapeDtypeStruct((E, B * K, D), jnp.float32),
    )(x_sel, gates, Wg, Wu, Wd)


def kernel(x, gate_weight, Wg, Wu, Wd):
    aff_t = _gating(x, gate_weight)
    x2d = x.reshape(B * S, D)
    gidx_eb, gates_eb, x_sel = _route_gather(aff_t, x2d)
    contrib = _swiglu(x_sel, gates_eb.reshape(E, 1, B * K), Wg, Wu, Wd)
    out = jnp.zeros((B * S, D), jnp.float32).at[gidx_eb.reshape(-1)].add(
        contrib.reshape(-1, D))
    return out.reshape(B, S, D)
